# baseline (speedup 1.0000x reference)
import jax
import jax.numpy as jnp
from jax import lax
from jax.experimental import pallas as pl
from jax.experimental.pallas import tpu as pltpu
from jax.experimental.pallas import tpu_sc as plsc

N_NODES = 10000
D = 1024
E = 25000
N_REL = 3
NC = 2
NS = 16
NW = NC * NS
BLK = 512
E_PAD = 25088
NBV = BLK // 16
NBLK = E_PAD // BLK
RPT = 80
SPAN = NW * RPT
N_PASS = 4
CAP = 1024
SPILL = CAP - BLK
PAD_DST = N_REL * N_NODES
TRASH = RPT

_CP = pltpu.CompilerParams(needs_layout_passes=False)


def _sc_body(x_all, src_all, dst_all, agg, dg,
             acc, dga, rows_v, srcblk, dstblk, srcf, ldstf, ssem, gsem):
    c = lax.axis_index("c")
    s = lax.axis_index("s")
    w = s * NC + c

    zero16 = jnp.zeros((16,), jnp.float32)
    one16 = jnp.ones((16,), jnp.float32)

    def _flush(cnt):
        srcf[pl.ds(cnt, 16)] = jnp.zeros((16,), jnp.int32)
        ldstf[pl.ds(cnt, 16)] = jnp.full((16,), TRASH, jnp.int32)
        nb = (cnt + 15) // 16

        def _j(j, _):
            pltpu.async_copy(x_all.at[srcf.at[pl.ds(j * 16, 16)]],
                             rows_v, gsem).wait()
            ld16 = ldstf[pl.ds(j * 16, 16)]
            for i in range(16):
                ri = ld16[i]
                for k in range(64):
                    off = k * 16
                    plsc.addupdate(acc.at[ri, pl.ds(off, 16)],
                                   rows_v[i, pl.ds(off, 16)])
                plsc.addupdate(dga.at[ri, pl.ds(0, 16)], one16)
            return 0
        lax.fori_loop(0, nb, _j, 0)

    def _step(q, _):
        rel = q // N_PASS
        p = q % N_PASS
        local_lo = p * SPAN + w * RPT
        lo_w = rel * N_NODES + local_lo

        @pl.when(local_lo < N_NODES)
        def _():
            def _z(rr, _):
                for k in range(64):
                    acc[rr, pl.ds(k * 16, 16)] = zero16
                dga[rr, pl.ds(0, 16)] = zero16
                return 0
            lax.fori_loop(0, RPT + 1, _z, 0)

            for pb in range(2):
                pltpu.async_copy(
                    src_all.at[pl.ds(rel * E_PAD + pb * BLK, BLK)],
                    srcblk.at[pl.ds(pb * BLK, BLK)], ssem)
                pltpu.async_copy(
                    dst_all.at[pl.ds(rel * E_PAD + pb * BLK, BLK)],
                    dstblk.at[pl.ds(pb * BLK, BLK)], ssem)

            def _blk(b, pos):
                bp = b % 3
                bp2 = (b + 2) % 3
                eoff = rel * E_PAD + b * BLK

                @pl.when(b + 2 < NBLK)
                def _():
                    pltpu.async_copy(
                        src_all.at[pl.ds(eoff + 2 * BLK, BLK)],
                        srcblk.at[pl.ds(bp2 * BLK, BLK)], ssem)
                    pltpu.async_copy(
                        dst_all.at[pl.ds(eoff + 2 * BLK, BLK)],
                        dstblk.at[pl.ds(bp2 * BLK, BLK)], ssem)

                @pl.when(pos >= SPILL)
                def _():
                    _flush(pos)
                pos = jnp.where(pos >= SPILL, 0, pos)

                pltpu.make_async_copy(src_all.at[pl.ds(eoff, BLK)],
                                      srcblk.at[pl.ds(bp * BLK, BLK)], ssem).wait()
                pltpu.make_async_copy(dst_all.at[pl.ds(eoff, BLK)],
                                      dstblk.at[pl.ds(bp * BLK, BLK)], ssem).wait()

                sb = srcblk.at[pl.ds(bp * BLK, BLK)]
                db = dstblk.at[pl.ds(bp * BLK, BLK)]

                def _filt(i, pos):
                    d16a = db[pl.ds(i * 32, 16)]
                    s16a = sb[pl.ds(i * 32, 16)]
                    d16b = db[pl.ds(i * 32 + 16, 16)]
                    s16b = sb[pl.ds(i * 32 + 16, 16)]
                    lda = d16a - lo_w
                    ldb = d16b - lo_w
                    ma = (lda >= 0) & (lda < RPT)
                    mb = (ldb >= 0) & (ldb < RPT)
                    na = plsc.all_reduce_population_count(ma)[0]
                    nb_ = plsc.all_reduce_population_count(mb)[0]
                    slota = pos + plsc.cumsum(ma.astype(jnp.int32)) - 1
                    slotb = (pos + na) + plsc.cumsum(mb.astype(jnp.int32)) - 1
                    plsc.store_scatter(srcf, [slota], s16a, mask=ma)
                    plsc.store_scatter(ldstf, [slota], lda, mask=ma)
                    plsc.store_scatter(srcf, [slotb], s16b, mask=mb)
                    plsc.store_scatter(ldstf, [slotb], ldb, mask=mb)
                    return pos + na + nb_
                return lax.fori_loop(0, NBV // 2, _filt, pos)
            pos = lax.fori_loop(0, NBLK, _blk, jnp.int32(0))
            _flush(pos)

            pltpu.sync_copy(acc.at[pl.ds(0, RPT)], agg.at[pl.ds(lo_w, RPT)])
            pltpu.sync_copy(dga.at[pl.ds(0, RPT)], dg.at[pl.ds(lo_w, RPT)])
        return 0
    lax.fori_loop(0, N_REL * N_PASS, _step, 0)


def _sc_aggregate(x_all, src_all, dst_all):
    mesh = plsc.VectorSubcoreMesh(core_axis_name="c", subcore_axis_name="s",
                                  num_cores=NC, num_subcores=NS)
    f32 = jnp.float32
    out_type = (
        jax.ShapeDtypeStruct((N_REL * N_NODES, D), f32),
        jax.ShapeDtypeStruct((N_REL * N_NODES, 16), f32),
    )
    scratch = [
        pltpu.VMEM((RPT + 1, D), f32),
        pltpu.VMEM((RPT + 1, 16), f32),
        pltpu.VMEM((16, D), f32),
        pltpu.VMEM((3 * BLK,), jnp.int32),
        pltpu.VMEM((3 * BLK,), jnp.int32),
        pltpu.VMEM((CAP + 16,), jnp.int32),
        pltpu.VMEM((CAP + 16,), jnp.int32),
        pltpu.SemaphoreType.DMA,
        pltpu.SemaphoreType.DMA,
    ]
    k = pl.kernel(_sc_body, out_type=out_type, mesh=mesh,
                  compiler_params=_CP, scratch_types=scratch)
    return k(x_all, src_all, dst_all)


_BR = 400


def _tc_body(aggd, degd, aggp1, degp1, aggp2, degp2,
             wd, bd, wp1, bp1, wp2, bp2, od, op):
    def nrm(a, g):
        deg = g[..., 0:1]
        n = jnp.where(deg > 0, 1.0 / deg, 0.0)
        return a[...] * n
    od[...] = jnp.dot(nrm(aggd, degd), wd[...],
                      preferred_element_type=jnp.float32) + bd[...]
    op[...] = (jnp.dot(nrm(aggp1, degp1), wp1[...],
                       preferred_element_type=jnp.float32)
               + jnp.dot(nrm(aggp2, degp2), wp2[...],
                         preferred_element_type=jnp.float32)
               + bp1[...] + bp2[...])


def _tc_finish(agg, dg, W_ddi, b_ddi, W_dpi, b_dpi, W_ppi, b_ppi):
    f32 = jnp.float32
    nb = N_NODES // _BR
    row0 = pl.BlockSpec((_BR, D), lambda i: (i, 0))
    row1 = pl.BlockSpec((_BR, D), lambda i: (i + nb, 0))
    row2 = pl.BlockSpec((_BR, D), lambda i: (i + 2 * nb, 0))
    dg0 = pl.BlockSpec((_BR, 16), lambda i: (i, 0))
    dg1 = pl.BlockSpec((_BR, 16), lambda i: (i + nb, 0))
    dg2 = pl.BlockSpec((_BR, 16), lambda i: (i + 2 * nb, 0))
    wsp = pl.BlockSpec((D, D), lambda i: (0, 0))
    bsp = pl.BlockSpec((1, D), lambda i: (0, 0))
    return pl.pallas_call(
        _tc_body,
        grid=(nb,),
        in_specs=[row0, dg0, row1, dg1, row2, dg2, wsp, bsp, wsp, bsp, wsp, bsp],
        out_specs=[pl.BlockSpec((_BR, D), lambda i: (i, 0)),
                   pl.BlockSpec((_BR, D), lambda i: (i, 0))],
        out_shape=[jax.ShapeDtypeStruct((N_NODES, D), f32),
                   jax.ShapeDtypeStruct((N_NODES, D), f32)],
    )(agg, dg, agg, dg, agg, dg,
      W_ddi, b_ddi.reshape(1, D), W_dpi, b_dpi.reshape(1, D),
      W_ppi, b_ppi.reshape(1, D))


def kernel(x_drug, x_protein, edge_index_ddi, edge_index_dpi, edge_index_ppi,
           W_ddi, b_ddi, W_dpi, b_dpi, W_ppi, b_ppi):
    x_all = jnp.concatenate([x_drug, x_protein], axis=0)
    npad = E_PAD - E

    def pad_edges(e, src_off, dst_off):
        src = jnp.concatenate([e[0] + src_off, jnp.zeros((npad,), jnp.int32)])
        dst = jnp.concatenate([e[1] + dst_off,
                               jnp.full((npad,), PAD_DST, jnp.int32)])
        return src, dst

    s0, d0 = pad_edges(edge_index_ddi, 0, 0)
    s1, d1 = pad_edges(edge_index_dpi, 0, N_NODES)
    s2, d2 = pad_edges(edge_index_ppi, N_NODES, 2 * N_NODES)
    src_all = jnp.concatenate([s0, s1, s2])
    dst_all = jnp.concatenate([d0, d1, d2])

    agg, dg = _sc_aggregate(x_all, src_all, dst_all)
    out_drug, out_protein = _tc_finish(
        agg, dg, W_ddi, b_ddi, W_dpi, b_dpi, W_ppi, b_ppi)
    return (out_drug, out_protein)


# speedup vs baseline: 1.1173x; 1.1173x over previous
import jax
import jax.numpy as jnp
from jax import lax
from jax.experimental import pallas as pl
from jax.experimental.pallas import tpu as pltpu
from jax.experimental.pallas import tpu_sc as plsc

N_NODES = 10000
D = 1024
E = 25000
N_REL = 3
NC = 2
NS = 16
NW = NC * NS
BLK = 512
E_PAD = 25088
NBV = BLK // 16
NBLK = E_PAD // BLK
RPT = 80
SPAN = NW * RPT
N_PASS = 4
CAP = 1024
SPILL = CAP - BLK
PAD_DST = N_REL * N_NODES

_CP = pltpu.CompilerParams(needs_layout_passes=False)


def _sc_body(x_all, src_all, dst_all, agg, dg,
             acc, dga, rows_v, srcblk, dstblk, srcf, ldstf, ssem, gsem):
    c = lax.axis_index("c")
    s = lax.axis_index("s")
    w = s * NC + c

    zero16 = jnp.zeros((16,), jnp.float32)
    one16 = jnp.ones((16,), jnp.float32)

    def _flush(cnt):
        srcf[pl.ds(cnt, 16)] = jnp.zeros((16,), jnp.int32)
        nb = (cnt + 15) // 16

        def _j(j, _):
            jp = j % 2

            @pl.when(j < nb)
            def _():
                sidx = srcf[pl.ds(j * 16, 16)]
                pltpu.async_copy(x_all.at[sidx], rows_v.at[pl.ds(jp * 16, 16)], gsem)

            @pl.when(j > 0)
            def _():
                jm = j - 1
                rv = rows_v.at[pl.ds((1 - jp) * 16, 16)]
                ld16 = ldstf[pl.ds(jm * 16, 16)]
                for i in range(16):
                    @pl.when(jm * 16 + i < cnt)
                    def _(i=i):
                        ri = ld16[i]
                        for k in range(64):
                            off = k * 16
                            plsc.addupdate(acc.at[ri, pl.ds(off, 16)],
                                           rv[i, pl.ds(off, 16)])
                        plsc.addupdate(dga.at[ri, pl.ds(0, 16)], one16)

            @pl.when(j < nb)
            def _():
                sidx = srcf[pl.ds(j * 16, 16)]
                pltpu.make_async_copy(x_all.at[sidx], rows_v.at[pl.ds(jp * 16, 16)],
                                      gsem).wait()
            return 0
        lax.fori_loop(0, nb + 1, _j, 0)

    def _step(q, _):
        rel = q // N_PASS
        p = q % N_PASS
        local_lo = p * SPAN + w * RPT
        lo_w = rel * N_NODES + local_lo

        @pl.when(local_lo < N_NODES)
        def _():
            def _z(rr, _):
                for k in range(64):
                    acc[rr, pl.ds(k * 16, 16)] = zero16
                dga[rr, pl.ds(0, 16)] = zero16
                return 0
            lax.fori_loop(0, RPT, _z, 0)

            pltpu.async_copy(src_all.at[pl.ds(rel * E_PAD, BLK)],
                             srcblk.at[pl.ds(0, BLK)], ssem)
            pltpu.async_copy(dst_all.at[pl.ds(rel * E_PAD, BLK)],
                             dstblk.at[pl.ds(0, BLK)], ssem)

            def _blk(b, pos):
                bp = b % 2
                eoff = rel * E_PAD + b * BLK

                @pl.when(b + 1 < NBLK)
                def _():
                    pltpu.async_copy(
                        src_all.at[pl.ds(eoff + BLK, BLK)],
                        srcblk.at[pl.ds((1 - bp) * BLK, BLK)], ssem)
                    pltpu.async_copy(
                        dst_all.at[pl.ds(eoff + BLK, BLK)],
                        dstblk.at[pl.ds((1 - bp) * BLK, BLK)], ssem)

                @pl.when(pos >= SPILL)
                def _():
                    _flush(pos)
                pos = jnp.where(pos >= SPILL, 0, pos)

                pltpu.make_async_copy(src_all.at[pl.ds(eoff, BLK)],
                                      srcblk.at[pl.ds(bp * BLK, BLK)], ssem).wait()
                pltpu.make_async_copy(dst_all.at[pl.ds(eoff, BLK)],
                                      dstblk.at[pl.ds(bp * BLK, BLK)], ssem).wait()

                sb = srcblk.at[pl.ds(bp * BLK, BLK)]
                db = dstblk.at[pl.ds(bp * BLK, BLK)]

                def _filt(i, pos):
                    d16a = db[pl.ds(i * 32, 16)]
                    s16a = sb[pl.ds(i * 32, 16)]
                    d16b = db[pl.ds(i * 32 + 16, 16)]
                    s16b = sb[pl.ds(i * 32 + 16, 16)]
                    lda = d16a - lo_w
                    ldb = d16b - lo_w
                    ma = (lda >= 0) & (lda < RPT)
                    mb = (ldb >= 0) & (ldb < RPT)
                    na = plsc.all_reduce_population_count(ma)[0]
                    nb_ = plsc.all_reduce_population_count(mb)[0]
                    slota = pos + plsc.cumsum(ma.astype(jnp.int32)) - 1
                    slotb = (pos + na) + plsc.cumsum(mb.astype(jnp.int32)) - 1
                    plsc.store_scatter(srcf, [slota], s16a, mask=ma)
                    plsc.store_scatter(ldstf, [slota], lda, mask=ma)
                    plsc.store_scatter(srcf, [slotb], s16b, mask=mb)
                    plsc.store_scatter(ldstf, [slotb], ldb, mask=mb)
                    return pos + na + nb_
                return lax.fori_loop(0, NBV // 2, _filt, pos)
            pos = lax.fori_loop(0, NBLK, _blk, jnp.int32(0))
            _flush(pos)

            pltpu.sync_copy(acc, agg.at[pl.ds(lo_w, RPT)])
            pltpu.sync_copy(dga, dg.at[pl.ds(lo_w, RPT)])
        return 0
    lax.fori_loop(0, N_REL * N_PASS, _step, 0)


def _sc_aggregate(x_all, src_all, dst_all):
    mesh = plsc.VectorSubcoreMesh(core_axis_name="c", subcore_axis_name="s",
                                  num_cores=NC, num_subcores=NS)
    f32 = jnp.float32
    out_type = (
        jax.ShapeDtypeStruct((N_REL * N_NODES, D), f32),
        jax.ShapeDtypeStruct((N_REL * N_NODES, 16), f32),
    )
    scratch = [
        pltpu.VMEM((RPT, D), f32),
        pltpu.VMEM((RPT, 16), f32),
        pltpu.VMEM((32, D), f32),
        pltpu.VMEM((2 * BLK,), jnp.int32),
        pltpu.VMEM((2 * BLK,), jnp.int32),
        pltpu.VMEM((CAP + 16,), jnp.int32),
        pltpu.VMEM((CAP + 16,), jnp.int32),
        pltpu.SemaphoreType.DMA,
        pltpu.SemaphoreType.DMA,
    ]
    k = pl.kernel(_sc_body, out_type=out_type, mesh=mesh,
                  compiler_params=_CP, scratch_types=scratch)
    return k(x_all, src_all, dst_all)


_BR = 400


def _tc_body(aggd, degd, aggp1, degp1, aggp2, degp2,
             wd, bd, wp1, bp1, wp2, bp2, od, op):
    def nrm(a, g):
        deg = g[..., 0:1]
        n = jnp.where(deg > 0, 1.0 / deg, 0.0)
        return a[...] * n
    od[...] = jnp.dot(nrm(aggd, degd), wd[...],
                      preferred_element_type=jnp.float32) + bd[...]
    op[...] = (jnp.dot(nrm(aggp1, degp1), wp1[...],
                       preferred_element_type=jnp.float32)
               + jnp.dot(nrm(aggp2, degp2), wp2[...],
                         preferred_element_type=jnp.float32)
               + bp1[...] + bp2[...])


def _tc_finish(agg, dg, W_ddi, b_ddi, W_dpi, b_dpi, W_ppi, b_ppi):
    f32 = jnp.float32
    nb = N_NODES // _BR
    row0 = pl.BlockSpec((_BR, D), lambda i: (i, 0))
    row1 = pl.BlockSpec((_BR, D), lambda i: (i + nb, 0))
    row2 = pl.BlockSpec((_BR, D), lambda i: (i + 2 * nb, 0))
    dg0 = pl.BlockSpec((_BR, 16), lambda i: (i, 0))
    dg1 = pl.BlockSpec((_BR, 16), lambda i: (i + nb, 0))
    dg2 = pl.BlockSpec((_BR, 16), lambda i: (i + 2 * nb, 0))
    wsp = pl.BlockSpec((D, D), lambda i: (0, 0))
    bsp = pl.BlockSpec((1, D), lambda i: (0, 0))
    return pl.pallas_call(
        _tc_body,
        grid=(nb,),
        in_specs=[row0, dg0, row1, dg1, row2, dg2, wsp, bsp, wsp, bsp, wsp, bsp],
        out_specs=[pl.BlockSpec((_BR, D), lambda i: (i, 0)),
                   pl.BlockSpec((_BR, D), lambda i: (i, 0))],
        out_shape=[jax.ShapeDtypeStruct((N_NODES, D), f32),
                   jax.ShapeDtypeStruct((N_NODES, D), f32)],
    )(agg, dg, agg, dg, agg, dg,
      W_ddi, b_ddi.reshape(1, D), W_dpi, b_dpi.reshape(1, D),
      W_ppi, b_ppi.reshape(1, D))


def kernel(x_drug, x_protein, edge_index_ddi, edge_index_dpi, edge_index_ppi,
           W_ddi, b_ddi, W_dpi, b_dpi, W_ppi, b_ppi):
    x_all = jnp.concatenate([x_drug, x_protein], axis=0)
    npad = E_PAD - E

    def pad_edges(e, src_off, dst_off):
        src = jnp.concatenate([e[0] + src_off, jnp.zeros((npad,), jnp.int32)])
        dst = jnp.concatenate([e[1] + dst_off,
                               jnp.full((npad,), PAD_DST, jnp.int32)])
        return src, dst

    s0, d0 = pad_edges(edge_index_ddi, 0, 0)
    s1, d1 = pad_edges(edge_index_dpi, 0, N_NODES)
    s2, d2 = pad_edges(edge_index_ppi, N_NODES, 2 * N_NODES)
    src_all = jnp.concatenate([s0, s1, s2])
    dst_all = jnp.concatenate([d0, d1, d2])

    agg, dg = _sc_aggregate(x_all, src_all, dst_all)
    out_drug, out_protein = _tc_finish(
        agg, dg, W_ddi, b_ddi, W_dpi, b_dpi, W_ppi, b_ppi)
    return (out_drug, out_protein)
